# baseline (speedup 1.0000x reference)
import jax
import jax.numpy as jnp
from jax import lax
from jax.experimental import pallas as pl
from jax.experimental.pallas import tpu as pltpu
from jax.experimental.pallas import tpu_sc as plsc

B, N, D = 16, 4096, 11
R = 512
MAX_ATOMS = 24
RES_PER_W = R // 2
PTS_FLAT = N * D
PTS_PAD = PTS_FLAT + 16
N_GROUPS = RES_PER_W // 16
NB_ROWS = D * MAX_ATOMS


def _body(pts_hbm, rs_hbm, nb_hbm, cen_hbm, msk_hbm,
          pts_v, starts_v, nb_v, cxb, cyb, czb, msk_v):
    b = lax.axis_index("s")
    half = lax.axis_index("c")
    r0 = half * RES_PER_W

    pltpu.sync_copy(pts_hbm.at[pl.ds(b * PTS_FLAT, PTS_FLAT)],
                    pts_v.at[pl.ds(0, PTS_FLAT)])
    pltpu.sync_copy(rs_hbm.at[pl.ds(b * R, R)], starts_v.at[pl.ds(0, R)])
    starts_v[pl.ds(R, 16)] = jnp.full((16,), N, dtype=jnp.int32)

    zero16 = jnp.zeros((16,), jnp.float32)

    def group(g, carry):
        gbase = r0 + g * 16
        sv = starts_v[pl.ds(gbase, 16)]
        ev = starts_v[pl.ds(gbase + 1, 16)]
        cntv = ev - sv
        maxc = jnp.max(cntv)

        def abody(j, acc):
            ax, ay, az = acc
            idx = sv + j
            m = idx < ev
            base = jnp.minimum(idx, N - 1)
            gx = plsc.load_gather(pts_v, [base], mask=m)
            gy = plsc.load_gather(pts_v, [base + N], mask=m)
            gz = plsc.load_gather(pts_v, [base + 2 * N], mask=m)
            ax = ax + jnp.where(m, gx, 0.0)
            ay = ay + jnp.where(m, gy, 0.0)
            az = az + jnp.where(m, gz, 0.0)
            return ax, ay, az

        ax, ay, az = lax.fori_loop(0, maxc, abody, (zero16, zero16, zero16))
        validv = cntv > 0
        invv = 1.0 / jnp.maximum(cntv, 1).astype(jnp.float32)
        cxv = jnp.where(validv, ax * invv, 0.0)
        cyv = jnp.where(validv, ay * invv, 0.0)
        czv = jnp.where(validv, az * invv, 0.0)

        o = g * 16
        cxb[pl.ds(o, 16)] = cxv
        cyb[pl.ds(o, 16)] = cyv
        czb[pl.ds(o, 16)] = czv
        msk_v[pl.ds(o, 16)] = jnp.where(validv, 1.0, 0.0)
        return carry

    lax.fori_loop(0, N_GROUPS, group, 0)

    def window(g):
        gbase = r0 + g * 16
        o = g * 16
        sv = starts_v[pl.ds(gbase, 16)]
        ev = starts_v[pl.ds(gbase + 1, 16)]
        cntv = ev - sv
        cens = (cxb[pl.ds(o, 16)], cyb[pl.ds(o, 16)], czb[pl.ds(o, 16)])
        for j in range(MAX_ATOMS):
            mj = j < cntv
            bvj = jnp.minimum(sv + j, N - 1)
            for d in range(D):
                gv = plsc.load_gather(pts_v, [bvj + d * N], mask=mj)
                if d < 3:
                    gv = gv - cens[d]
                outv = jnp.where(mj, gv, 0.0)
                nb_v[d * MAX_ATOMS + j, pl.ds(o, 16)] = outv

    plsc.parallel_loop(0, N_GROUPS, 1, unroll=2)(window)

    pltpu.sync_copy(nb_v, nb_hbm.at[pl.ds(b * NB_ROWS, NB_ROWS),
                                    pl.ds(r0, RES_PER_W)])
    pltpu.sync_copy(cxb, cen_hbm.at[pl.ds(b * R + r0, RES_PER_W)])
    pltpu.sync_copy(cyb, cen_hbm.at[pl.ds((16 + b) * R + r0, RES_PER_W)])
    pltpu.sync_copy(czb, cen_hbm.at[pl.ds((32 + b) * R + r0, RES_PER_W)])
    pltpu.sync_copy(msk_v, msk_hbm.at[pl.ds(b * R + r0, RES_PER_W)])


@jax.jit
def _run(pts_flat, residue_starts_flat):
    mesh = plsc.VectorSubcoreMesh(core_axis_name="c", subcore_axis_name="s",
                                  num_cores=2, num_subcores=16)
    f = pl.kernel(
        _body,
        out_type=(
            jax.ShapeDtypeStruct((B * NB_ROWS, R), jnp.float32),
            jax.ShapeDtypeStruct((3 * B * R,), jnp.float32),
            jax.ShapeDtypeStruct((B * R,), jnp.float32),
        ),
        mesh=mesh,
        compiler_params=pltpu.CompilerParams(needs_layout_passes=False),
        scratch_types=[
            pltpu.VMEM((PTS_PAD,), jnp.float32),
            pltpu.VMEM((R + 16,), jnp.int32),
            pltpu.VMEM((NB_ROWS, RES_PER_W), jnp.float32),
            pltpu.VMEM((RES_PER_W,), jnp.float32),
            pltpu.VMEM((RES_PER_W,), jnp.float32),
            pltpu.VMEM((RES_PER_W,), jnp.float32),
            pltpu.VMEM((RES_PER_W,), jnp.float32),
        ],
    )
    return f(pts_flat, residue_starts_flat)


def kernel(points, residue_starts):
    pts_t = points.transpose(0, 2, 1).reshape(B * D * N)
    nb, cen, msk = _run(pts_t, residue_starts.reshape(B * R))
    nb = nb.reshape(B, D, MAX_ATOMS, R).transpose(0, 3, 2, 1)
    cen = cen.reshape(3, B, R).transpose(1, 2, 0)
    return (nb, cen, msk.reshape(B, R))
